# baseline (speedup 1.0000x reference)
import jax
import jax.numpy as jnp
from jax.experimental import pallas as pl
from jax.experimental.pallas import tpu as pltpu
from jax.experimental.pallas import tpu_sc as plsc

_B = 16
_SC_CORES = 2
_SC_SUBCORES = 16
_UNITS = _SC_CORES * _SC_SUBCORES
_CHUNK = 16
_LANES = 16
_TC_ROWS = 28672
_TC_BLK = 2048


def _sc_partial_sums(flat, ids_r, row_base):
    total, d = flat.shape
    nunits, nchunks, _ = ids_r.shape
    rows_per_unit = nchunks * _CHUNK
    mesh = plsc.VectorSubcoreMesh(core_axis_name="c", subcore_axis_name="s")

    @pl.kernel(
        out_type=jax.ShapeDtypeStruct((_UNITS, _B, d), jnp.float32),
        mesh=mesh,
        scratch_types=[
            pltpu.VMEM((nchunks, _CHUNK), jnp.int32),
            pltpu.VMEM((_CHUNK, d), jnp.float32),
            pltpu.VMEM((_CHUNK, d), jnp.float32),
            pltpu.VMEM((_B, d), jnp.float32),
            pltpu.SemaphoreType.DMA,
            pltpu.SemaphoreType.DMA,
            pltpu.SemaphoreType.DMA,
        ],
    )
    def sc_kernel(flat_hbm, ids_hbm, out_hbm, ids_v, buf_a, buf_b, acc,
                  sem_a, sem_b, sem_i):
        c = jax.lax.axis_index("c")
        s = jax.lax.axis_index("s")
        u = c * _SC_SUBCORES + s
        ids_cp = pltpu.async_copy(ids_hbm.at[u], ids_v, sem_i)

        row0_pre = row_base + u * rows_per_unit
        cp_pre_a = pltpu.async_copy(
            flat_hbm.at[pl.ds(row0_pre, _CHUNK)], buf_a, sem_a)
        cp_pre_b = pltpu.async_copy(
            flat_hbm.at[pl.ds(row0_pre + _CHUNK, _CHUNK)], buf_b, sem_b)

        zeros = jnp.zeros((_LANES,), jnp.float32)

        @pl.loop(0, _B)
        def _(r):
            @pl.loop(0, d, step=_LANES)
            def _(col):
                acc[r, pl.ds(col, _LANES)] = zeros

        row0 = row_base + u * rows_per_unit

        def process(buf, k):
            idvec = ids_v[k]
            first = idvec[0]
            same = first == idvec[_CHUNK - 1]

            @pl.when(same)
            def _fast():
                @pl.loop(0, d, step=2 * _LANES)
                def _(col):
                    cols = [col, col + _LANES]
                    trees = []
                    for cc in cols:
                        xs = [buf[r, pl.ds(cc, _LANES)]
                              for r in range(_CHUNK)]
                        trees.append(xs)
                    vals = []
                    for xs in trees:
                        while len(xs) > 1:
                            pairs = [xs[i] + xs[i + 1]
                                     for i in range(0, len(xs) - 1, 2)]
                            if len(xs) % 2:
                                pairs.append(xs[-1])
                            xs = pairs
                        vals.append(xs[0])
                    for cc, v in zip(cols, vals):
                        plsc.addupdate(acc.at[first, pl.ds(cc, _LANES)], v)

            @pl.when(jnp.logical_not(same))
            def _slow():
                for r in range(_CHUNK):
                    seg = idvec[r]

                    @pl.loop(0, d, step=_LANES)
                    def _(col):
                        plsc.addupdate(acc.at[seg, pl.ds(col, _LANES)],
                                       buf[r, pl.ds(col, _LANES)])

        ids_cp.wait()
        cp_pre_a.wait()
        process(buf_a, 0)

        @pl.loop(1, nchunks - 1, step=2)
        def _(kb):
            cp_a = pltpu.async_copy(
                flat_hbm.at[pl.ds(row0 + (kb + 1) * _CHUNK, _CHUNK)], buf_a,
                sem_a)
            pltpu.make_async_copy(
                flat_hbm.at[pl.ds(row0 + kb * _CHUNK, _CHUNK)], buf_b,
                sem_b).wait()
            process(buf_b, kb)
            cp_b = pltpu.async_copy(
                flat_hbm.at[pl.ds(row0 + (kb + 2) * _CHUNK, _CHUNK)], buf_b,
                sem_b)
            cp_a.wait()
            process(buf_a, kb + 1)

        pltpu.make_async_copy(
            flat_hbm.at[pl.ds(row0 + (nchunks - 1) * _CHUNK, _CHUNK)], buf_b,
            sem_b).wait()
        process(buf_b, nchunks - 1)

        pltpu.async_copy(acc, out_hbm.at[u], sem_a).wait()

    return sc_kernel(flat, ids_r)


def _tc_partial_body(ids_ref, x_ref, out_ref):
    i = pl.program_id(0)

    @pl.when(i == 0)
    def _init():
        out_ref[...] = jnp.zeros_like(out_ref)

    ids = ids_ref[0, 0, :]
    onehot = (
        ids[:, None] == jax.lax.broadcasted_iota(jnp.int32, (_TC_BLK, _B), 1)
    ).astype(jnp.float32)
    out_ref[...] += jax.lax.dot_general(
        onehot, x_ref[...], (((0,), (0,)), ((), ())),
        preferred_element_type=jnp.float32,
    )


def _tc_partial_sums(flat, ids_tc):
    d = flat.shape[1]
    n_blocks = _TC_ROWS // _TC_BLK
    ids3 = ids_tc.reshape(n_blocks, 1, _TC_BLK)
    return pl.pallas_call(
        _tc_partial_body,
        grid=(n_blocks,),
        in_specs=[
            pl.BlockSpec((1, 1, _TC_BLK), lambda i: (i, 0, 0)),
            pl.BlockSpec((_TC_BLK, d), lambda i: (i, 0)),
        ],
        out_specs=pl.BlockSpec((_B, d), lambda i: (0, 0)),
        out_shape=jax.ShapeDtypeStruct((_B, d), jnp.float32),
        compiler_params=pltpu.CompilerParams(
            dimension_semantics=("arbitrary",),
        ),
    )(ids3, flat)


def _combine_body(ids_ref, tcp_ref, scp_ref, out_ref):
    total = ids_ref.shape[1]
    counts = jnp.sum(
        (ids_ref[0][None, :]
         == jax.lax.broadcasted_iota(jnp.int32, (_B, total), 0)
         ).astype(jnp.float32),
        axis=1,
    )
    sums = tcp_ref[...] + jnp.sum(scp_ref[...], axis=0)
    out_ref[...] = sums / jnp.maximum(counts, 1.0)[:, None]


def _combine(tc_partial, sc_partials, ids):
    total = ids.shape[0]
    units, b, d = sc_partials.shape
    return pl.pallas_call(
        _combine_body,
        in_specs=[
            pl.BlockSpec((1, total), lambda: (0, 0)),
            pl.BlockSpec((b, d), lambda: (0, 0)),
            pl.BlockSpec((units, b, d), lambda: (0, 0, 0)),
        ],
        out_specs=pl.BlockSpec((b, d), lambda: (0, 0)),
        out_shape=jax.ShapeDtypeStruct((b, d), jnp.float32),
    )(ids.reshape(1, total), tc_partial, sc_partials)


def kernel(flat, segment_ids):
    ids = segment_ids.astype(jnp.int32)
    ids_sc = ids[_TC_ROWS:].reshape(_UNITS, -1, _CHUNK)
    sc_partials = _sc_partial_sums(flat, ids_sc, _TC_ROWS)
    tc_partial = _tc_partial_sums(flat, ids[:_TC_ROWS])
    return _combine(tc_partial, sc_partials, ids)


# speedup vs baseline: 1.0202x; 1.0202x over previous
import jax
import jax.numpy as jnp
from jax.experimental import pallas as pl
from jax.experimental.pallas import tpu as pltpu
from jax.experimental.pallas import tpu_sc as plsc

_B = 16
_SC_CORES = 2
_SC_SUBCORES = 16
_UNITS = _SC_CORES * _SC_SUBCORES
_CHUNK = 16
_LANES = 16
_TC_ROWS = 28672
_TC_BLK = 1024


def _sc_partial_sums(flat, ids_r, row_base):
    total, d = flat.shape
    nunits, nchunks, _ = ids_r.shape
    rows_per_unit = nchunks * _CHUNK
    mesh = plsc.VectorSubcoreMesh(core_axis_name="c", subcore_axis_name="s")

    @pl.kernel(
        out_type=jax.ShapeDtypeStruct((_UNITS, _B, d), jnp.float32),
        mesh=mesh,
        scratch_types=[
            pltpu.VMEM((nchunks, _CHUNK), jnp.int32),
            pltpu.VMEM((_CHUNK, d), jnp.float32),
            pltpu.VMEM((_CHUNK, d), jnp.float32),
            pltpu.VMEM((_B, d), jnp.float32),
            pltpu.SemaphoreType.DMA,
            pltpu.SemaphoreType.DMA,
            pltpu.SemaphoreType.DMA,
        ],
    )
    def sc_kernel(flat_hbm, ids_hbm, out_hbm, ids_v, buf_a, buf_b, acc,
                  sem_a, sem_b, sem_i):
        c = jax.lax.axis_index("c")
        s = jax.lax.axis_index("s")
        u = c * _SC_SUBCORES + s
        ids_cp = pltpu.async_copy(ids_hbm.at[u], ids_v, sem_i)

        row0_pre = row_base + u * rows_per_unit
        cp_pre_a = pltpu.async_copy(
            flat_hbm.at[pl.ds(row0_pre, _CHUNK)], buf_a, sem_a)
        cp_pre_b = pltpu.async_copy(
            flat_hbm.at[pl.ds(row0_pre + _CHUNK, _CHUNK)], buf_b, sem_b)

        zeros = jnp.zeros((_LANES,), jnp.float32)

        @pl.loop(0, _B)
        def _(r):
            @pl.loop(0, d, step=_LANES)
            def _(col):
                acc[r, pl.ds(col, _LANES)] = zeros

        row0 = row_base + u * rows_per_unit

        def process(buf, k):
            idvec = ids_v[k]
            first = idvec[0]
            same = first == idvec[_CHUNK - 1]

            @pl.when(same)
            def _fast():
                @pl.loop(0, d, step=2 * _LANES)
                def _(col):
                    cols = [col, col + _LANES]
                    trees = []
                    for cc in cols:
                        xs = [buf[r, pl.ds(cc, _LANES)]
                              for r in range(_CHUNK)]
                        trees.append(xs)
                    vals = []
                    for xs in trees:
                        while len(xs) > 1:
                            pairs = [xs[i] + xs[i + 1]
                                     for i in range(0, len(xs) - 1, 2)]
                            if len(xs) % 2:
                                pairs.append(xs[-1])
                            xs = pairs
                        vals.append(xs[0])
                    for cc, v in zip(cols, vals):
                        plsc.addupdate(acc.at[first, pl.ds(cc, _LANES)], v)

            @pl.when(jnp.logical_not(same))
            def _slow():
                for r in range(_CHUNK):
                    seg = idvec[r]

                    @pl.loop(0, d, step=_LANES)
                    def _(col):
                        plsc.addupdate(acc.at[seg, pl.ds(col, _LANES)],
                                       buf[r, pl.ds(col, _LANES)])

        ids_cp.wait()
        cp_pre_a.wait()
        process(buf_a, 0)

        @pl.loop(1, nchunks - 1, step=2)
        def _(kb):
            cp_a = pltpu.async_copy(
                flat_hbm.at[pl.ds(row0 + (kb + 1) * _CHUNK, _CHUNK)], buf_a,
                sem_a)
            pltpu.make_async_copy(
                flat_hbm.at[pl.ds(row0 + kb * _CHUNK, _CHUNK)], buf_b,
                sem_b).wait()
            process(buf_b, kb)
            cp_b = pltpu.async_copy(
                flat_hbm.at[pl.ds(row0 + (kb + 2) * _CHUNK, _CHUNK)], buf_b,
                sem_b)
            cp_a.wait()
            process(buf_a, kb + 1)

        pltpu.make_async_copy(
            flat_hbm.at[pl.ds(row0 + (nchunks - 1) * _CHUNK, _CHUNK)], buf_b,
            sem_b).wait()
        process(buf_b, nchunks - 1)

        pltpu.async_copy(acc, out_hbm.at[u], sem_a).wait()

    return sc_kernel(flat, ids_r)


def _tc_partial_body(ids_ref, x_ref, out_ref):
    i = pl.program_id(0)

    @pl.when(i == 0)
    def _init():
        out_ref[...] = jnp.zeros_like(out_ref)

    ids = ids_ref[0, 0, :]
    onehot = (
        ids[:, None] == jax.lax.broadcasted_iota(jnp.int32, (_TC_BLK, _B), 1)
    ).astype(jnp.float32)
    out_ref[...] += jax.lax.dot_general(
        onehot, x_ref[...], (((0,), (0,)), ((), ())),
        preferred_element_type=jnp.float32,
    )


def _tc_partial_sums(flat, ids_tc):
    d = flat.shape[1]
    n_blocks = _TC_ROWS // _TC_BLK
    ids3 = ids_tc.reshape(n_blocks, 1, _TC_BLK)
    return pl.pallas_call(
        _tc_partial_body,
        grid=(n_blocks,),
        in_specs=[
            pl.BlockSpec((1, 1, _TC_BLK), lambda i: (i, 0, 0)),
            pl.BlockSpec((_TC_BLK, d), lambda i: (i, 0)),
        ],
        out_specs=pl.BlockSpec((_B, d), lambda i: (0, 0)),
        out_shape=jax.ShapeDtypeStruct((_B, d), jnp.float32),
        compiler_params=pltpu.CompilerParams(
            dimension_semantics=("arbitrary",),
        ),
    )(ids3, flat)


def _combine_body(ids_ref, tcp_ref, scp_ref, out_ref):
    total = ids_ref.shape[1]
    counts = jnp.sum(
        (ids_ref[0][None, :]
         == jax.lax.broadcasted_iota(jnp.int32, (_B, total), 0)
         ).astype(jnp.float32),
        axis=1,
    )
    sums = tcp_ref[...] + jnp.sum(scp_ref[...], axis=0)
    out_ref[...] = sums / jnp.maximum(counts, 1.0)[:, None]


def _combine(tc_partial, sc_partials, ids):
    total = ids.shape[0]
    units, b, d = sc_partials.shape
    return pl.pallas_call(
        _combine_body,
        in_specs=[
            pl.BlockSpec((1, total), lambda: (0, 0)),
            pl.BlockSpec((b, d), lambda: (0, 0)),
            pl.BlockSpec((units, b, d), lambda: (0, 0, 0)),
        ],
        out_specs=pl.BlockSpec((b, d), lambda: (0, 0)),
        out_shape=jax.ShapeDtypeStruct((b, d), jnp.float32),
    )(ids.reshape(1, total), tc_partial, sc_partials)


def kernel(flat, segment_ids):
    ids = segment_ids.astype(jnp.int32)
    ids_sc = ids[_TC_ROWS:].reshape(_UNITS, -1, _CHUNK)
    sc_partials = _sc_partial_sums(flat, ids_sc, _TC_ROWS)
    tc_partial = _tc_partial_sums(flat, ids[:_TC_ROWS])
    return _combine(tc_partial, sc_partials, ids)
